# Initial kernel scaffold; baseline (speedup 1.0000x reference)
import jax
import jax.numpy as jnp
from jax.experimental import pallas as pl


def kernel(x_0, x_1, adj_0_0, adj_1_0, adj_1_1, inv_0_0, inv_1_0, inv_1_1, msg_W1_0_0, msg_b1_0_0, msg_W2_0_0, msg_b2_0_0, inf_W_0_0, inf_b_0_0, msg_W1_1_0, msg_b1_1_0, msg_W2_1_0, msg_b2_1_0, inf_W_1_0, inf_b_1_0, msg_W1_1_1, msg_b1_1_1, msg_W2_1_1, msg_b2_1_1, inf_W_1_1, inf_b_1_1, upd_W1_0, upd_b1_0, upd_W2_0, upd_b2_0, upd_W1_1, upd_b1_1, upd_W2_1, upd_b2_1):
    raise NotImplementedError("write your pallas kernel here")


# speedup vs baseline: 1.7079x; 1.7079x over previous
import functools

import jax
import jax.numpy as jnp
from jax import lax
from jax.experimental import pallas as pl
from jax.experimental.pallas import tpu as pltpu
from jax.experimental.pallas import tpu_sc as plsc

_N0, _N1, _E, _H, _NI = 10000, 20000, 160000, 256, 16
_NC, _NS = 2, 16
_NW = _NC * _NS
_EB = 128
_NBLK = _E // _EB
_NBF = _NBLK // _NW
_NBR = _NBLK - _NBF * _NW
_SBF = _NBLK // _NS
_SBR = _NBLK - _SBF * _NS

_f32 = jnp.float32


def _silu(x):
    return x * jax.nn.sigmoid(x)


def _mesh():
    return plsc.VectorSubcoreMesh(
        core_axis_name="c", subcore_axis_name="s",
        num_cores=_NC, num_subcores=_NS)


def _proj(x, ws, bs, br=1024):
    n = x.shape[0]
    nw = len(ws)

    def body(x_ref, *refs):
        w_refs = refs[:nw]
        b_refs = refs[nw:2 * nw]
        o_refs = refs[2 * nw:]
        xb = x_ref[...]
        for j in range(nw):
            o_refs[j][...] = (
                jnp.dot(xb, w_refs[j][...], preferred_element_type=_f32)
                + b_refs[j][...])

    in_specs = [pl.BlockSpec((br, _H), lambda i: (i, 0))]
    in_specs += [pl.BlockSpec((_H, _H), lambda i: (0, 0))] * nw
    in_specs += [pl.BlockSpec((1, _H), lambda i: (0, 0))] * nw
    out_specs = [pl.BlockSpec((br, _H), lambda i: (i, 0))] * nw
    return pl.pallas_call(
        body, grid=(pl.cdiv(n, br),),
        in_specs=in_specs, out_specs=out_specs,
        out_shape=[jax.ShapeDtypeStruct((n, _H), _f32)] * nw,
    )(x, *ws, *bs)


def _edge_mlp(ga, gb, inv, w1i, w2, b2, wip, bip, be=1280):
    def body(ga_ref, gb_ref, inv_ref, w1i_ref, w2_ref, b2_ref, wip_ref,
             bip_ref, y_ref):
        t = (ga_ref[...] + gb_ref[...]
             + jnp.dot(inv_ref[...], w1i_ref[...], preferred_element_type=_f32))
        m = _silu(t)
        m2 = _silu(jnp.dot(m, w2_ref[...], preferred_element_type=_f32)
                   + b2_ref[...])
        g = jax.nn.sigmoid(
            jnp.dot(m2, wip_ref[...], preferred_element_type=_f32)
            + bip_ref[...])
        y_ref[...] = m2 * g[:, 0:1]

    in_specs = [
        pl.BlockSpec((be, _H), lambda i: (i, 0)),
        pl.BlockSpec((be, _H), lambda i: (i, 0)),
        pl.BlockSpec((be, _NI), lambda i: (i, 0)),
        pl.BlockSpec((_NI, _H), lambda i: (0, 0)),
        pl.BlockSpec((_H, _H), lambda i: (0, 0)),
        pl.BlockSpec((1, _H), lambda i: (0, 0)),
        pl.BlockSpec((_H, 128), lambda i: (0, 0)),
        pl.BlockSpec((1, 128), lambda i: (0, 0)),
    ]
    return pl.pallas_call(
        body, grid=(_E // be,),
        in_specs=in_specs,
        out_specs=pl.BlockSpec((be, _H), lambda i: (i, 0)),
        out_shape=jax.ShapeDtypeStruct((_E, _H), _f32),
    )(ga, gb, inv, w1i, w2, b2, wip, bip)


def _sc_gather(tab_a, idx0, tab_b, idx1):
    @functools.partial(
        pl.kernel,
        out_type=[jax.ShapeDtypeStruct((_E, _H), _f32)] * 2,
        mesh=_mesh(),
        scratch_types=[
            pltpu.VMEM((_EB,), jnp.int32),
            pltpu.VMEM((_EB,), jnp.int32),
            pltpu.VMEM((_EB, _H), _f32),
            pltpu.VMEM((_EB, _H), _f32),
            pltpu.SemaphoreType.DMA,
            pltpu.SemaphoreType.DMA,
        ],
    )
    def k(a_hbm, i0_hbm, b_hbm, i1_hbm, ga_hbm, gb_hbm,
          idxa, idxb, rowsa, rowsb, sema, semb):
        w = lax.axis_index("s") * _NC + lax.axis_index("c")

        def block(j, carry):
            e0 = pl.multiple_of((w + j * _NW) * _EB, _EB)
            pltpu.sync_copy(i0_hbm.at[pl.ds(e0, _EB)], idxa)
            pltpu.sync_copy(i1_hbm.at[pl.ds(e0, _EB)], idxb)
            ca = pltpu.async_copy(a_hbm.at[idxa], rowsa, sema)
            cb = pltpu.async_copy(b_hbm.at[idxb], rowsb, semb)
            ca.wait()
            cb.wait()
            pltpu.sync_copy(rowsa, ga_hbm.at[pl.ds(e0, _EB)])
            pltpu.sync_copy(rowsb, gb_hbm.at[pl.ds(e0, _EB)])
            return carry

        lax.fori_loop(0, _NBF, block, 0)

        @pl.when(w < _NBR)
        def _():
            block(_NBF, 0)

    return k(tab_a, idx0, tab_b, idx1)


def _sc_scatter_n0(y0, i0, y1, i1, zeros):
    rt = _N0 // _NS
    cw = _H // _NC

    @functools.partial(
        pl.kernel,
        out_type=[jax.ShapeDtypeStruct((_N0, _H), _f32)] * 2,
        mesh=_mesh(),
        scratch_types=[
            pltpu.VMEM((_EB,), jnp.int32),
            pltpu.VMEM((_EB, cw), _f32),
            pltpu.VMEM_SHARED((_N0, cw), _f32),
        ],
        compiler_params=pltpu.CompilerParams(use_tc_tiling_on_sc=False),
    )
    def k(z_hbm, y0_hbm, i0_hbm, y1_hbm, i1_hbm, m0_hbm, m1_hbm,
          idxv, ybuf, accum):
        core = lax.axis_index("c")
        sub = lax.axis_index("s")
        col0 = core * cw
        r0 = sub * rt

        for y_hbm, i_hbm, m_hbm in ((y0_hbm, i0_hbm, m0_hbm),
                                    (y1_hbm, i1_hbm, m1_hbm)):
            pltpu.sync_copy(z_hbm.at[pl.ds(0, rt), pl.ds(0, cw)],
                            accum.at[pl.ds(r0, rt)])
            plsc.subcore_barrier()

            def block(j, carry):
                e0 = pl.multiple_of((sub + j * _NS) * _EB, _EB)
                pltpu.sync_copy(i_hbm.at[pl.ds(e0, _EB)], idxv)
                pltpu.sync_copy(y_hbm.at[pl.ds(e0, _EB), pl.ds(col0, cw)],
                                ybuf)
                pltpu.sync_copy(ybuf, accum.at[idxv], add=True)
                return carry

            lax.fori_loop(0, _SBF, block, 0)

            @pl.when(sub < _SBR)
            def _():
                block(_SBF, 0)

            plsc.subcore_barrier()
            pltpu.sync_copy(accum.at[pl.ds(r0, rt)],
                            m_hbm.at[pl.ds(r0, rt), pl.ds(col0, cw)])
            plsc.subcore_barrier()

    return k(zeros, y0, i0, y1, i1)


def _sc_scatter_n1(y, i, zeros):
    rt = _N1 // _NS
    cw = _H // (2 * _NC)

    @functools.partial(
        pl.kernel,
        out_type=jax.ShapeDtypeStruct((_N1, _H), _f32),
        mesh=_mesh(),
        scratch_types=[
            pltpu.VMEM((_EB,), jnp.int32),
            pltpu.VMEM((_EB, cw), _f32),
            pltpu.VMEM_SHARED((_N1, cw), _f32),
        ],
        compiler_params=pltpu.CompilerParams(use_tc_tiling_on_sc=False),
    )
    def k(z_hbm, y_hbm, i_hbm, m_hbm, idxv, ybuf, accum):
        core = lax.axis_index("c")
        sub = lax.axis_index("s")
        r0 = sub * rt

        for p in range(2):
            col0 = core * 2 * cw + p * cw
            pltpu.sync_copy(z_hbm.at[pl.ds(0, rt), pl.ds(0, cw)],
                            accum.at[pl.ds(r0, rt)])
            plsc.subcore_barrier()

            def block(j, carry):
                e0 = pl.multiple_of((sub + j * _NS) * _EB, _EB)
                pltpu.sync_copy(i_hbm.at[pl.ds(e0, _EB)], idxv)
                pltpu.sync_copy(y_hbm.at[pl.ds(e0, _EB), pl.ds(col0, cw)],
                                ybuf)
                pltpu.sync_copy(ybuf, accum.at[idxv], add=True)
                return carry

            lax.fori_loop(0, _SBF, block, 0)

            @pl.when(sub < _SBR)
            def _():
                block(_SBF, 0)

            plsc.subcore_barrier()
            pltpu.sync_copy(accum.at[pl.ds(r0, rt)],
                            m_hbm.at[pl.ds(r0, rt), pl.ds(col0, cw)])
            plsc.subcore_barrier()

    return k(zeros, y, i)


def _update(parts, w_parts, b1, w2, b2, br=1024):
    n = parts[0].shape[0]
    k = len(parts)

    def body(*refs):
        p_refs = refs[:k]
        wp_refs = refs[k:2 * k]
        b1_ref, w2_ref, b2_ref, o_ref = refs[2 * k:]
        s = jnp.dot(p_refs[0][...], wp_refs[0][...],
                    preferred_element_type=_f32)
        for j in range(1, k):
            s = s + jnp.dot(p_refs[j][...], wp_refs[j][...],
                            preferred_element_type=_f32)
        h = _silu(s + b1_ref[...])
        o_ref[...] = (jnp.dot(h, w2_ref[...], preferred_element_type=_f32)
                      + b2_ref[...] + p_refs[0][...])

    in_specs = [pl.BlockSpec((br, _H), lambda i: (i, 0))] * k
    in_specs += [pl.BlockSpec((_H, _H), lambda i: (0, 0))] * k
    in_specs += [pl.BlockSpec((1, _H), lambda i: (0, 0)),
                 pl.BlockSpec((_H, _H), lambda i: (0, 0)),
                 pl.BlockSpec((1, _H), lambda i: (0, 0))]
    return pl.pallas_call(
        body, grid=(pl.cdiv(n, br),),
        in_specs=in_specs,
        out_specs=pl.BlockSpec((br, _H), lambda i: (i, 0)),
        out_shape=jax.ShapeDtypeStruct((n, _H), _f32),
    )(*parts, *w_parts, b1, w2, b2)


def kernel(x_0, x_1, adj_0_0, adj_1_0, adj_1_1, inv_0_0, inv_1_0, inv_1_1,
           msg_W1_0_0, msg_b1_0_0, msg_W2_0_0, msg_b2_0_0, inf_W_0_0, inf_b_0_0,
           msg_W1_1_0, msg_b1_1_0, msg_W2_1_0, msg_b2_1_0, inf_W_1_0, inf_b_1_0,
           msg_W1_1_1, msg_b1_1_1, msg_W2_1_1, msg_b2_1_1, inf_W_1_1, inf_b_1_1,
           upd_W1_0, upd_b1_0, upd_W2_0, upd_b2_0,
           upd_W1_1, upd_b1_1, upd_W2_1, upd_b2_1):
    i32 = jnp.int32
    idx = {
        "0_0": (adj_0_0[0].astype(i32), adj_0_0[1].astype(i32)),
        "1_0": (adj_1_0[0].astype(i32), adj_1_0[1].astype(i32)),
        "1_1": (adj_1_1[0].astype(i32), adj_1_1[1].astype(i32)),
    }
    w1 = {"0_0": msg_W1_0_0, "1_0": msg_W1_1_0, "1_1": msg_W1_1_1}
    b1 = {"0_0": msg_b1_0_0, "1_0": msg_b1_1_0, "1_1": msg_b1_1_1}
    w2 = {"0_0": msg_W2_0_0, "1_0": msg_W2_1_0, "1_1": msg_W2_1_1}
    b2 = {"0_0": msg_b2_0_0, "1_0": msg_b2_1_0, "1_1": msg_b2_1_1}
    wi = {"0_0": inf_W_0_0, "1_0": inf_W_1_0, "1_1": inf_W_1_1}
    bi = {"0_0": inf_b_0_0, "1_0": inf_b_1_0, "1_1": inf_b_1_1}
    inv = {"0_0": inv_0_0, "1_0": inv_1_0, "1_1": inv_1_1}

    w1s = {a: w1[a][:_H] for a in w1}
    w1r = {a: w1[a][_H:2 * _H] for a in w1}
    w1i = {a: w1[a][2 * _H:] for a in w1}
    b1row = {a: b1[a][None, :] for a in b1}
    b2row = {a: b2[a][None, :] for a in b2}
    wip = {a: jnp.pad(wi[a], ((0, 0), (0, 127))) for a in wi}
    bip = {a: jnp.pad(bi[a], (0, 127))[None, :] for a in bi}
    zrow = jnp.zeros((1, _H), _f32)

    A00, B00, B10 = _proj(
        x_0, [w1s["0_0"], w1r["0_0"], w1r["1_0"]],
        [zrow, b1row["0_0"], b1row["1_0"]])
    A10, A11, B11 = _proj(
        x_1, [w1s["1_0"], w1s["1_1"], w1r["1_1"]],
        [zrow, zrow, b1row["1_1"]])
    tabs = {"0_0": (A00, B00), "1_0": (A10, B10), "1_1": (A11, B11)}

    y = {}
    for a in ("0_0", "1_0", "1_1"):
        i0, i1 = idx[a]
        ga, gb = _sc_gather(tabs[a][0], i0, tabs[a][1], i1)
        y[a] = _edge_mlp(ga, gb, inv[a], w1i[a], w2[a], b2row[a],
                         wip[a], bip[a])

    zeros = jnp.zeros((_N1 // _NS, _H // _NC), _f32)
    mes00, mes10 = _sc_scatter_n0(y["0_0"], idx["0_0"][1],
                                  y["1_0"], idx["1_0"][1], zeros)
    mes11 = _sc_scatter_n1(y["1_1"], idx["1_1"][1], zeros)

    u1_0 = [upd_W1_0[:_H], upd_W1_0[_H:2 * _H], upd_W1_0[2 * _H:]]
    out0 = _update([x_0, mes00, mes10], u1_0, upd_b1_0[None, :],
                   upd_W2_0, upd_b2_0[None, :])
    u1_1 = [upd_W1_1[:_H], upd_W1_1[_H:]]
    out1 = _update([x_1, mes11], u1_1, upd_b1_1[None, :],
                   upd_W2_1, upd_b2_1[None, :])
    return (out0, out1)
